# baseline (speedup 1.0000x reference)
import jax
from jax.experimental import pallas as pl
from jax.experimental.pallas import tpu as pltpu


_SPLITS = 4


def _dma_copy_kernel(u, i, t, ou, oi, ot, su, si, st):
    copies = []
    for src, dst, sem in ((u, ou, su), (i, oi, si), (t, ot, st)):
        rows = src.shape[0] // _SPLITS
        for k in range(_SPLITS):
            c = pltpu.make_async_copy(
                src.at[pl.ds(k * rows, rows)],
                dst.at[pl.ds(k * rows, rows)],
                sem,
            )
            c.start()
            copies.append(c)
    for c in copies:
        c.wait()


def kernel(embed_user, embed_item, embed_tag):
    hbm_spec = pl.BlockSpec(memory_space=pltpu.MemorySpace.HBM)
    return pl.pallas_call(
        _dma_copy_kernel,
        in_specs=[hbm_spec, hbm_spec, hbm_spec],
        out_specs=[hbm_spec, hbm_spec, hbm_spec],
        scratch_shapes=[pltpu.SemaphoreType.DMA] * 3,
        out_shape=[
            jax.ShapeDtypeStruct(embed_user.shape, embed_user.dtype),
            jax.ShapeDtypeStruct(embed_item.shape, embed_item.dtype),
            jax.ShapeDtypeStruct(embed_tag.shape, embed_tag.dtype),
        ],
    )(embed_user, embed_item, embed_tag)


# speedup vs baseline: 48.0518x; 48.0518x over previous
import jax
from jax.experimental import pallas as pl
from jax.experimental.pallas import tpu as pltpu


_TC_STEPS = 25


def _copy3_kernel(u_ref, i_ref, t_ref, ou_ref, oi_ref, ot_ref):
    ou_ref[...] = u_ref[...]
    oi_ref[...] = i_ref[...]
    ot_ref[...] = t_ref[...]


def kernel(embed_user, embed_item, embed_tag):
    nu, d = embed_user.shape
    ni, _ = embed_item.shape
    nt, _ = embed_tag.shape
    bu, bi, bt = nu // _TC_STEPS, ni // _TC_STEPS, nt // _TC_STEPS

    def spec(block_rows):
        return pl.BlockSpec((block_rows, d), lambda s: (s, 0))

    return pl.pallas_call(
        _copy3_kernel,
        grid=(_TC_STEPS,),
        compiler_params=pltpu.CompilerParams(dimension_semantics=("parallel",)),
        in_specs=[spec(bu), spec(bi), spec(bt)],
        out_specs=[spec(bu), spec(bi), spec(bt)],
        out_shape=[
            jax.ShapeDtypeStruct(embed_user.shape, embed_user.dtype),
            jax.ShapeDtypeStruct(embed_item.shape, embed_item.dtype),
            jax.ShapeDtypeStruct(embed_tag.shape, embed_tag.dtype),
        ],
    )(embed_user, embed_item, embed_tag)
